# Initial kernel scaffold; baseline (speedup 1.0000x reference)
import jax
import jax.numpy as jnp
from jax.experimental import pallas as pl


def kernel(points, W1, b1, W2, b2, W3, b3, W4, b4, Wm, bm, Wc1, bc1, Wc2, bc2, Wc3, bc3):
    raise NotImplementedError("write your pallas kernel here")


# speedup vs baseline: 6.2422x; 6.2422x over previous
import functools

import jax
import jax.numpy as jnp
from jax import lax
from jax.experimental import pallas as pl
from jax.experimental.pallas import tpu as pltpu

_K = 20


def _leaky(x):
    return jnp.where(x > 0, x, 0.2 * x)


def _edgeconv_body(x_ref, xt_ref, w_ref, b_ref, out_ref, sq_scr, *, R, N):
    r = pl.program_id(1)

    @pl.when(r == 0)
    def _():
        xc = xt_ref[0]
        sq_scr[0:1, :] = jnp.sum(xc * xc, axis=0, keepdims=True)

    rows = x_ref[0, pl.ds(r * R, R), :]
    inner = lax.dot_general(rows.astype(jnp.bfloat16), xt_ref[0].astype(jnp.bfloat16),
                            (((1,), (0,)), ((), ())),
                            preferred_element_type=jnp.float32)
    sq_m = sq_scr[0:1, :]
    eyer = (lax.broadcasted_iota(jnp.int32, (R, R), 0)
            == lax.broadcasted_iota(jnp.int32, (R, R), 1)).astype(jnp.float32)
    sq_r = lax.dot_general(eyer, sq_scr[0:1, pl.ds(r * R, R)],
                           (((1,), (1,)), ((), ())),
                           preferred_element_type=jnp.float32,
                           precision=lax.Precision.HIGHEST)
    s = (2.0 * inner - sq_r) - sq_m
    iota = lax.broadcasted_iota(jnp.int32, (R, N), 1)

    xf = x_ref[0]
    x_hi = xf.astype(jnp.bfloat16)
    r1 = xf - x_hi.astype(jnp.float32)
    x_mid = r1.astype(jnp.bfloat16)
    x_lo = (r1 - x_mid.astype(jnp.float32)).astype(jnp.bfloat16)

    wb = w_ref[...].astype(jnp.bfloat16)
    O = wb.shape[1]
    acc = jnp.full((R, O), -jnp.inf, jnp.float32)
    for _ in range(_K):
        mval = jnp.max(s, axis=1, keepdims=True)
        eq = s == mval
        cand = jnp.min(jnp.where(eq, iota, N), axis=1, keepdims=True)
        oh = iota == cand
        ohb = oh.astype(jnp.bfloat16)
        dims = (((1,), (0,)), ((), ()))
        nbr = (lax.dot_general(ohb, x_hi, dims, preferred_element_type=jnp.float32)
               + lax.dot_general(ohb, x_mid, dims, preferred_element_type=jnp.float32)
               + lax.dot_general(ohb, x_lo, dims, preferred_element_type=jnp.float32))
        edge = jnp.concatenate([nbr - rows, rows], axis=1).astype(jnp.bfloat16)
        h = lax.dot_general(edge, wb, dims, preferred_element_type=jnp.float32) + b_ref[...]
        acc = jnp.maximum(acc, _leaky(h))
        s = jnp.where(oh, -jnp.inf, s)
    out_ref[0] = acc


def _edgeconv(x, xt, W, b, R=256):
    B, N, C = x.shape
    O = W.shape[1]
    return pl.pallas_call(
        functools.partial(_edgeconv_body, R=R, N=N),
        grid=(B, N // R),
        in_specs=[
            pl.BlockSpec((1, N, C), lambda b_, r_: (b_, 0, 0)),
            pl.BlockSpec((1, C, N), lambda b_, r_: (b_, 0, 0)),
            pl.BlockSpec((2 * C, O), lambda b_, r_: (0, 0)),
            pl.BlockSpec((1, O), lambda b_, r_: (0, 0)),
        ],
        out_specs=pl.BlockSpec((1, R, O), lambda b_, r_: (b_, r_, 0)),
        out_shape=jax.ShapeDtypeStruct((B, N, O), jnp.float32),
        scratch_shapes=[
            pltpu.VMEM((8, N), jnp.float32),
        ],
    )(x, xt, W, b.reshape(1, O))


def _pool_body(p1_ref, p2_ref, p3_ref, p4_ref, w1_ref, w2_ref, w3_ref, w4_ref,
               bm_ref, mx_ref, av_ref, acc_ref, *, NR, N):
    r = pl.program_id(1)
    h = (jnp.dot(p1_ref[0], w1_ref[...], preferred_element_type=jnp.float32, precision=lax.Precision.HIGHEST)
         + jnp.dot(p2_ref[0], w2_ref[...], preferred_element_type=jnp.float32, precision=lax.Precision.HIGHEST)
         + jnp.dot(p3_ref[0], w3_ref[...], preferred_element_type=jnp.float32, precision=lax.Precision.HIGHEST)
         + jnp.dot(p4_ref[0], w4_ref[...], preferred_element_type=jnp.float32, precision=lax.Precision.HIGHEST)
         + bm_ref[...])
    h = _leaky(h)
    bmax = jnp.max(h, axis=0, keepdims=True)
    bsum = jnp.sum(h, axis=0, keepdims=True)

    @pl.when(r == 0)
    def _():
        acc_ref[0:1, :] = bmax
        acc_ref[1:2, :] = bsum

    @pl.when(r != 0)
    def _():
        acc_ref[0:1, :] = jnp.maximum(acc_ref[0:1, :], bmax)
        acc_ref[1:2, :] = acc_ref[1:2, :] + bsum

    @pl.when(r == NR - 1)
    def _():
        mx_ref[0] = acc_ref[0:1, :]
        av_ref[0] = acc_ref[1:2, :] * (1.0 / N)


def _pool(p1, p2, p3, p4, Wm, bm, Rp=512):
    B, N, _ = p1.shape
    H = Wm.shape[1]
    NR = N // Rp
    w1 = Wm[:64]
    w2 = Wm[64:128]
    w3 = Wm[128:256]
    w4 = Wm[256:512]
    return pl.pallas_call(
        functools.partial(_pool_body, NR=NR, N=N),
        grid=(B, NR),
        in_specs=[
            pl.BlockSpec((1, Rp, 64), lambda b_, r_: (b_, r_, 0)),
            pl.BlockSpec((1, Rp, 64), lambda b_, r_: (b_, r_, 0)),
            pl.BlockSpec((1, Rp, 128), lambda b_, r_: (b_, r_, 0)),
            pl.BlockSpec((1, Rp, 256), lambda b_, r_: (b_, r_, 0)),
            pl.BlockSpec((64, H), lambda b_, r_: (0, 0)),
            pl.BlockSpec((64, H), lambda b_, r_: (0, 0)),
            pl.BlockSpec((128, H), lambda b_, r_: (0, 0)),
            pl.BlockSpec((256, H), lambda b_, r_: (0, 0)),
            pl.BlockSpec((1, H), lambda b_, r_: (0, 0)),
        ],
        out_specs=[
            pl.BlockSpec((1, 1, H), lambda b_, r_: (b_, 0, 0)),
            pl.BlockSpec((1, 1, H), lambda b_, r_: (b_, 0, 0)),
        ],
        out_shape=[
            jax.ShapeDtypeStruct((B, 1, H), jnp.float32),
            jax.ShapeDtypeStruct((B, 1, H), jnp.float32),
        ],
        scratch_shapes=[pltpu.VMEM((8, H), jnp.float32)],
    )(p1, p2, p3, p4, w1, w2, w3, w4, bm.reshape(1, H))


def _cls_body(mx_ref, av_ref, a1_ref, a2_ref, bc1_ref, w2_ref, bc2_ref,
              w3_ref, bc3_ref, out_ref):
    f1 = _leaky(jnp.dot(mx_ref[...], a1_ref[...], preferred_element_type=jnp.float32, precision=lax.Precision.HIGHEST)
                + jnp.dot(av_ref[...], a2_ref[...], preferred_element_type=jnp.float32, precision=lax.Precision.HIGHEST)
                + bc1_ref[...])
    f2 = _leaky(jnp.dot(f1, w2_ref[...], preferred_element_type=jnp.float32, precision=lax.Precision.HIGHEST)
                + bc2_ref[...])
    out_ref[...] = (jnp.dot(f2, w3_ref[...], preferred_element_type=jnp.float32, precision=lax.Precision.HIGHEST)
                    + bc3_ref[...])


def _classifier(mx, av, Wc1, bc1, Wc2, bc2, Wc3, bc3):
    B, H = mx.shape
    a1 = Wc1[:H]
    a2 = Wc1[H:]
    C1 = Wc1.shape[1]
    C2 = Wc2.shape[1]
    C3 = Wc3.shape[1]
    return pl.pallas_call(
        _cls_body,
        out_shape=jax.ShapeDtypeStruct((B, C3), jnp.float32),
    )(mx, av, a1, a2, bc1.reshape(1, C1), Wc2, bc2.reshape(1, C2),
      Wc3, bc3.reshape(1, C3))


def kernel(points, W1, b1, W2, b2, W3, b3, W4, b4, Wm, bm,
           Wc1, bc1, Wc2, bc2, Wc3, bc3):
    x = jnp.transpose(points, (0, 2, 1))
    p1 = _edgeconv(x, points, W1, b1)
    p2 = _edgeconv(p1, jnp.transpose(p1, (0, 2, 1)), W2, b2)
    p3 = _edgeconv(p2, jnp.transpose(p2, (0, 2, 1)), W3, b3)
    p4 = _edgeconv(p3, jnp.transpose(p3, (0, 2, 1)), W4, b4)
    mx, av = _pool(p1, p2, p3, p4, Wm, bm)
    mx = mx[:, 0, :]
    av = av[:, 0, :]
    return _classifier(mx, av, Wc1, bc1, Wc2, bc2, Wc3, bc3)
